# Initial kernel scaffold; baseline (speedup 1.0000x reference)
import functools

import jax
import jax.numpy as jnp
from jax import lax
from jax.experimental import pallas as pl
from jax.experimental.pallas import tpu as pltpu
from jax.experimental.pallas import tpu_sc as plsc

N = 10000
E = 320000
D = 128
DE = 16
CUTOFF = 5.0

NC = 2
NS = 16
NW = NC * NS


def _silu(x):
    return x * jax.nn.sigmoid(x)


def _a1_body(ns_ref, w1_ref, b1_ref, w2_ref, b2_ref, so_ref):
    h = _silu(jnp.dot(ns_ref[...], w1_ref[...],
                      preferred_element_type=jnp.float32) + b1_ref[...])
    so_ref[...] = jnp.dot(h, w2_ref[...],
                          preferred_element_type=jnp.float32) + b2_ref[...]


def _a2_body(es_ref, norms_ref, wf_ref, bf_ref, fw_ref):
    fw = jnp.dot(es_ref[...], wf_ref[...],
                 preferred_element_type=jnp.float32) + bf_ref[...]
    r = norms_ref[...]
    cut = jnp.where(r < CUTOFF,
                    0.5 * (jnp.cos(jnp.pi * r / CUTOFF) + 1.0), 0.0)
    fw_ref[...] = fw * cut


def _m_body(fw_ref, sog_ref, nsvg_ref, ev_ref, msg_ref):
    fo = fw_ref[...] * sog_ref[...]
    gn = fo[:, 0:D]
    ge = fo[:, D:2 * D]
    msg_ref[0] = fo[:, 2 * D:3 * D]
    nsvg = nsvg_ref[...]
    ev = ev_ref[...]
    for k in range(3):
        msg_ref[1 + k] = nsvg[:, k * D:(k + 1) * D] * gn + ge * ev[:, k:k + 1]


def _b_body(ns_ref, nsv_ref, dp_ref, u_ref, v_ref, wa1_ref, ba1_ref,
            wa2_ref, ba2_ref, outs_ref, outv_ref):
    ns2 = ns_ref[...] + dp_ref[0]
    nv = [nsv_ref[:, k * D:(k + 1) * D] + dp_ref[1 + k] for k in range(3)]
    u = u_ref[...]
    v = v_ref[...]
    uv = [jnp.dot(nv[k], u, preferred_element_type=jnp.float32)
          for k in range(3)]
    vv = [jnp.dot(nv[k], v, preferred_element_type=jnp.float32)
          for k in range(3)]
    vv_sq = vv[0] * vv[0] + vv[1] * vv[1] + vv[2] * vv[2]
    inner = uv[0] * vv[0] + uv[1] * vv[1] + uv[2] * vv[2]
    wa1 = wa1_ref[...]
    h = (jnp.dot(ns2, wa1[0:D, :], preferred_element_type=jnp.float32)
         + jnp.dot(vv_sq, wa1[D:2 * D, :], preferred_element_type=jnp.float32)
         + ba1_ref[...])
    a = jnp.dot(_silu(h), wa2_ref[...],
                preferred_element_type=jnp.float32) + ba2_ref[...]
    outs_ref[...] = ns2 + a[:, 0:D] + a[:, D:2 * D] * inner
    for k in range(3):
        outv_ref[:, k * D:(k + 1) * D] = nv[k] + a[:, 2 * D:3 * D] * uv[k]


_MESH = plsc.VectorSubcoreMesh(core_axis_name="c", subcore_axis_name="s")

_G_EW = E // NW
_G_BLK = 80
_G_NB = _G_EW // _G_BLK

_S_EW = E // NS
_S_BLK = 80
_S_NB = _S_EW // _S_BLK
_N_SUB = N // NS


def _gather_rows_body(table_hbm, src_hbm, out_hbm, idx_v, buf0, buf1,
                      sem0, sem1):
    wid = lax.axis_index("s") * NC + lax.axis_index("c")
    base = wid * _G_EW
    pltpu.sync_copy(src_hbm.at[pl.ds(base, _G_EW)], idx_v)

    def gather(i, buf, sem):
        return pltpu.async_copy(
            table_hbm.at[idx_v.at[pl.ds(i * _G_BLK, _G_BLK)]], buf, sem)

    gather(0, buf0, sem0)

    @pl.loop(0, _G_NB)
    def _(i):
        nxt = i + 1

        @pl.when(nxt < _G_NB)
        def _():
            @pl.when(lax.rem(nxt, 2) == 0)
            def _():
                gather(nxt, buf0, sem0)

            @pl.when(lax.rem(nxt, 2) == 1)
            def _():
                gather(nxt, buf1, sem1)

        @pl.when(lax.rem(i, 2) == 0)
        def _():
            pltpu.make_async_copy(
                table_hbm.at[idx_v.at[pl.ds(i * _G_BLK, _G_BLK)]], buf0,
                sem0).wait()
            pltpu.sync_copy(buf0, out_hbm.at[pl.ds(base + i * _G_BLK, _G_BLK)])

        @pl.when(lax.rem(i, 2) == 1)
        def _():
            pltpu.make_async_copy(
                table_hbm.at[idx_v.at[pl.ds(i * _G_BLK, _G_BLK)]], buf1,
                sem1).wait()
            pltpu.sync_copy(buf1, out_hbm.at[pl.ds(base + i * _G_BLK, _G_BLK)])


def _gather_call(table, src):
    k = pl.kernel(
        _gather_rows_body,
        out_type=jax.ShapeDtypeStruct((E, table.shape[1]), jnp.float32),
        mesh=_MESH,
        scratch_types=[
            pltpu.VMEM((_G_EW,), jnp.int32),
            pltpu.VMEM((_G_BLK, table.shape[1]), jnp.float32),
            pltpu.VMEM((_G_BLK, table.shape[1]), jnp.float32),
            pltpu.SemaphoreType.DMA,
            pltpu.SemaphoreType.DMA,
        ],
    )
    return k(table, src)


def _scatter_body(msg_hbm, dst_hbm, zeros_hbm, dp_hbm, acc, idx_v, buf0, buf1,
                  sem0, sem1):
    core = lax.axis_index("c")
    sid = lax.axis_index("s")
    pltpu.sync_copy(dst_hbm.at[pl.ds(sid * _S_NB, _S_NB)], idx_v)

    for p_i in range(2):
        plane = 2 * core + p_i
        pltpu.sync_copy(zeros_hbm, acc.at[pl.ds(sid * _N_SUB, _N_SUB)])
        plsc.subcore_barrier()

        def load(i, buf, sem):
            return pltpu.async_copy(
                msg_hbm.at[plane, pl.ds(sid * _S_EW + i * _S_BLK, _S_BLK)],
                buf, sem)

        load(0, buf0, sem0)

        @pl.loop(0, _S_NB)
        def _(i):
            nxt = i + 1

            @pl.when(nxt < _S_NB)
            def _():
                @pl.when(lax.rem(nxt, 2) == 0)
                def _():
                    load(nxt, buf0, sem0)

                @pl.when(lax.rem(nxt, 2) == 1)
                def _():
                    load(nxt, buf1, sem1)

            @pl.when(lax.rem(i, 2) == 0)
            def _():
                pltpu.make_async_copy(
                    msg_hbm.at[plane,
                               pl.ds(sid * _S_EW + i * _S_BLK, _S_BLK)],
                    buf0, sem0).wait()
                pltpu.sync_copy(buf0, acc.at[idx_v.at[i]], add=True)

            @pl.when(lax.rem(i, 2) == 1)
            def _():
                pltpu.make_async_copy(
                    msg_hbm.at[plane,
                               pl.ds(sid * _S_EW + i * _S_BLK, _S_BLK)],
                    buf1, sem1).wait()
                pltpu.sync_copy(buf1, acc.at[idx_v.at[i]], add=True)

        plsc.subcore_barrier()
        pltpu.sync_copy(acc.at[pl.ds(sid * _N_SUB, _N_SUB)],
                        dp_hbm.at[plane, pl.ds(sid * _N_SUB, _N_SUB)])
        plsc.subcore_barrier()


def _scatter_call(msg, dst_r, zeros_sub):
    k = pl.kernel(
        _scatter_body,
        out_type=jax.ShapeDtypeStruct((4, N, D), jnp.float32),
        mesh=_MESH,
        scratch_types=[
            pltpu.VMEM_SHARED((N, D), jnp.float32),
            pltpu.VMEM((_S_NB, _S_BLK), jnp.int32),
            pltpu.VMEM((_S_BLK, D), jnp.float32),
            pltpu.VMEM((_S_BLK, D), jnp.float32),
            pltpu.SemaphoreType.DMA,
            pltpu.SemaphoreType.DMA,
        ],
    )
    return k(msg, dst_r, zeros_sub)


def kernel(node_states_scalar, node_states_vector, edge_states, edge_vectors,
           edge_norms, edge_index, Wf, bf, Ws1, bs1, Ws2, bs2, U, V,
           Wa1, ba1, Wa2, ba2):
    ns = node_states_scalar
    nsv_flat = node_states_vector.reshape(N, 3 * D)
    src = edge_index[:, 0]
    dst_r = edge_index[:, 1].reshape(E // _S_BLK, _S_BLK)
    zeros_sub = jnp.zeros((_N_SUB, D), jnp.float32)

    bn = 1000
    so = pl.pallas_call(
        _a1_body,
        grid=(N // bn,),
        in_specs=[
            pl.BlockSpec((bn, D), lambda i: (i, 0)),
            pl.BlockSpec((D, D), lambda i: (0, 0)),
            pl.BlockSpec((1, D), lambda i: (0, 0)),
            pl.BlockSpec((D, 3 * D), lambda i: (0, 0)),
            pl.BlockSpec((1, 3 * D), lambda i: (0, 0)),
        ],
        out_specs=pl.BlockSpec((bn, 3 * D), lambda i: (i, 0)),
        out_shape=jax.ShapeDtypeStruct((N, 3 * D), jnp.float32),
    )(ns, Ws1, bs1.reshape(1, D), Ws2, bs2.reshape(1, 3 * D))

    be = 4000
    fw = pl.pallas_call(
        _a2_body,
        grid=(E // be,),
        in_specs=[
            pl.BlockSpec((be, DE), lambda i: (i, 0)),
            pl.BlockSpec((be, 1), lambda i: (i, 0)),
            pl.BlockSpec((DE, 3 * D), lambda i: (0, 0)),
            pl.BlockSpec((1, 3 * D), lambda i: (0, 0)),
        ],
        out_specs=pl.BlockSpec((be, 3 * D), lambda i: (i, 0)),
        out_shape=jax.ShapeDtypeStruct((E, 3 * D), jnp.float32),
    )(edge_states, edge_norms, Wf, bf.reshape(1, 3 * D))

    nsv_g = _gather_call(nsv_flat, src)
    so_g = _gather_call(so, src)

    bm = 2000
    msg = pl.pallas_call(
        _m_body,
        grid=(E // bm,),
        in_specs=[
            pl.BlockSpec((bm, 3 * D), lambda i: (i, 0)),
            pl.BlockSpec((bm, 3 * D), lambda i: (i, 0)),
            pl.BlockSpec((bm, 3 * D), lambda i: (i, 0)),
            pl.BlockSpec((bm, 3), lambda i: (i, 0)),
        ],
        out_specs=pl.BlockSpec((4, bm, D), lambda i: (0, i, 0)),
        out_shape=jax.ShapeDtypeStruct((4, E, D), jnp.float32),
    )(fw, so_g, nsv_g, edge_vectors)

    dp = _scatter_call(msg, dst_r, zeros_sub)

    outs, outv = pl.pallas_call(
        _b_body,
        grid=(N // bn,),
        in_specs=[
            pl.BlockSpec((bn, D), lambda i: (i, 0)),
            pl.BlockSpec((bn, 3 * D), lambda i: (i, 0)),
            pl.BlockSpec((4, bn, D), lambda i: (0, i, 0)),
            pl.BlockSpec((D, D), lambda i: (0, 0)),
            pl.BlockSpec((D, D), lambda i: (0, 0)),
            pl.BlockSpec((2 * D, D), lambda i: (0, 0)),
            pl.BlockSpec((1, D), lambda i: (0, 0)),
            pl.BlockSpec((D, 3 * D), lambda i: (0, 0)),
            pl.BlockSpec((1, 3 * D), lambda i: (0, 0)),
        ],
        out_specs=[
            pl.BlockSpec((bn, D), lambda i: (i, 0)),
            pl.BlockSpec((bn, 3 * D), lambda i: (i, 0)),
        ],
        out_shape=[
            jax.ShapeDtypeStruct((N, D), jnp.float32),
            jax.ShapeDtypeStruct((N, 3 * D), jnp.float32),
        ],
    )(ns, nsv_flat, dp, U, V, Wa1, ba1.reshape(1, D), Wa2,
      ba2.reshape(1, 3 * D))

    return outs, outv.reshape(N, 3, D)


# speedup vs baseline: 17.8726x; 17.8726x over previous
import functools

import jax
import jax.numpy as jnp
from jax import lax
from jax.experimental import pallas as pl
from jax.experimental.pallas import tpu as pltpu
from jax.experimental.pallas import tpu_sc as plsc

N = 10000
E = 320000
D = 128
DE = 16
CUTOFF = 5.0

NC = 2
NS = 16
NW = NC * NS


def _silu(x):
    return x * jax.nn.sigmoid(x)


def _a1_body(ns_ref, w1_ref, b1_ref, w2_ref, b2_ref, so_ref):
    h = _silu(jnp.dot(ns_ref[...], w1_ref[...],
                      preferred_element_type=jnp.float32) + b1_ref[...])
    so_ref[...] = jnp.dot(h, w2_ref[...],
                          preferred_element_type=jnp.float32) + b2_ref[...]


def _a2_body(es_ref, norms_ref, wf_ref, bf_ref, fw_ref):
    fw = jnp.dot(es_ref[...], wf_ref[...],
                 preferred_element_type=jnp.float32) + bf_ref[...]
    r = norms_ref[...]
    cut = jnp.where(r < CUTOFF,
                    0.5 * (jnp.cos(jnp.pi * r / CUTOFF) + 1.0), 0.0)
    fw_ref[...] = fw * cut


def _m_body(fw_ref, sog_ref, nsvg_ref, ev_ref, msg_ref):
    fo = fw_ref[...] * sog_ref[...]
    gn = fo[:, 0:D]
    ge = fo[:, D:2 * D]
    msg_ref[0] = fo[:, 2 * D:3 * D]
    nsvg = nsvg_ref[...]
    ev = ev_ref[...]
    for k in range(3):
        msg_ref[1 + k] = nsvg[:, k * D:(k + 1) * D] * gn + ge * ev[:, k:k + 1]


def _b_body(ns_ref, nsv_ref, dp_ref, u_ref, v_ref, wa1_ref, ba1_ref,
            wa2_ref, ba2_ref, outs_ref, outv_ref):
    ns2 = ns_ref[...] + dp_ref[0]
    nv = [nsv_ref[:, k * D:(k + 1) * D] + dp_ref[1 + k] for k in range(3)]
    u = u_ref[...]
    v = v_ref[...]
    uv = [jnp.dot(nv[k], u, preferred_element_type=jnp.float32)
          for k in range(3)]
    vv = [jnp.dot(nv[k], v, preferred_element_type=jnp.float32)
          for k in range(3)]
    vv_sq = vv[0] * vv[0] + vv[1] * vv[1] + vv[2] * vv[2]
    inner = uv[0] * vv[0] + uv[1] * vv[1] + uv[2] * vv[2]
    wa1 = wa1_ref[...]
    h = (jnp.dot(ns2, wa1[0:D, :], preferred_element_type=jnp.float32)
         + jnp.dot(vv_sq, wa1[D:2 * D, :], preferred_element_type=jnp.float32)
         + ba1_ref[...])
    a = jnp.dot(_silu(h), wa2_ref[...],
                preferred_element_type=jnp.float32) + ba2_ref[...]
    outs_ref[...] = ns2 + a[:, 0:D] + a[:, D:2 * D] * inner
    for k in range(3):
        outv_ref[:, k * D:(k + 1) * D] = nv[k] + a[:, 2 * D:3 * D] * uv[k]


def _sc_mesh():
    return plsc.VectorSubcoreMesh(core_axis_name="c", subcore_axis_name="s",
                                  num_cores=NC, num_subcores=NS)

_G_EW = E // NW
_G_BLK = 80
_G_NB = _G_EW // _G_BLK

_S_BLK = 128
_EPAD = 2560 * _S_BLK
_S_EW = _EPAD // NS
_S_NB = _S_EW // _S_BLK
_S_CHUNK = 32
_NPAD = 10240
_N_SUB = _NPAD // NS


def _gather_rows_body(table_hbm, src_hbm, out_hbm, idx_v, buf0, buf1,
                      sem0, sem1):
    wid = lax.axis_index("s") * NC + lax.axis_index("c")
    base = wid * _G_EW
    pltpu.sync_copy(src_hbm.at[pl.ds(base, _G_EW)], idx_v)

    def gather(i, buf, sem):
        return pltpu.async_copy(
            table_hbm.at[idx_v.at[pl.ds(i * _G_BLK, _G_BLK)]], buf, sem)

    gather(0, buf0, sem0)

    @pl.loop(0, _G_NB)
    def _(i):
        nxt = i + 1

        @pl.when(nxt < _G_NB)
        def _():
            @pl.when(lax.rem(nxt, 2) == 0)
            def _():
                gather(nxt, buf0, sem0)

            @pl.when(lax.rem(nxt, 2) == 1)
            def _():
                gather(nxt, buf1, sem1)

        @pl.when(lax.rem(i, 2) == 0)
        def _():
            pltpu.make_async_copy(
                table_hbm.at[idx_v.at[pl.ds(i * _G_BLK, _G_BLK)]], buf0,
                sem0).wait()
            pltpu.sync_copy(buf0, out_hbm.at[pl.ds(base + i * _G_BLK, _G_BLK)])

        @pl.when(lax.rem(i, 2) == 1)
        def _():
            pltpu.make_async_copy(
                table_hbm.at[idx_v.at[pl.ds(i * _G_BLK, _G_BLK)]], buf1,
                sem1).wait()
            pltpu.sync_copy(buf1, out_hbm.at[pl.ds(base + i * _G_BLK, _G_BLK)])


def _gather_call(table, src):
    k = pl.kernel(
        _gather_rows_body,
        out_type=jax.ShapeDtypeStruct((E, table.shape[1]), jnp.float32),
        mesh=_sc_mesh(),
        scratch_types=[
            pltpu.VMEM((_G_EW,), jnp.int32),
            pltpu.VMEM((_G_BLK, table.shape[1]), jnp.float32),
            pltpu.VMEM((_G_BLK, table.shape[1]), jnp.float32),
            pltpu.SemaphoreType.DMA,
            pltpu.SemaphoreType.DMA,
        ],
    )
    return k(table, src)


def _scatter_body(msg_hbm, dst_hbm, zeros_hbm, dp_hbm, acc, idx_v, buf0, buf1,
                  sem0, sem1):
    core = lax.axis_index("c")
    sid = lax.axis_index("s")

    for p_i in range(2):
        plane = 2 * core + p_i
        pltpu.sync_copy(zeros_hbm, acc.at[pl.ds(sid * _N_SUB, _N_SUB)])
        plsc.subcore_barrier()

        def load(i, buf, sem):
            return pltpu.async_copy(
                msg_hbm.at[plane, pl.ds(sid * _S_EW + i * _S_BLK, _S_BLK)],
                buf, sem)

        @pl.loop(0, _S_NB // _S_CHUNK)
        def _(chunk):
            pltpu.sync_copy(
                dst_hbm.at[pl.ds(sid * _S_NB + chunk * _S_CHUNK, _S_CHUNK)],
                idx_v)
            base = chunk * _S_CHUNK
            load(base, buf0, sem0)

            @pl.loop(0, _S_CHUNK)
            def _(j):
                i = base + j
                nxt = i + 1

                @pl.when(jnp.logical_and(nxt < _S_NB, j + 1 < _S_CHUNK))
                def _():
                    @pl.when(lax.rem(nxt, 2) == 0)
                    def _():
                        load(nxt, buf0, sem0)

                    @pl.when(lax.rem(nxt, 2) == 1)
                    def _():
                        load(nxt, buf1, sem1)

                @pl.when(lax.rem(i, 2) == 0)
                def _():
                    pltpu.make_async_copy(
                        msg_hbm.at[plane,
                                   pl.ds(sid * _S_EW + i * _S_BLK, _S_BLK)],
                        buf0, sem0).wait()
                    pltpu.sync_copy(buf0, acc.at[idx_v.at[j]], add=True)

                @pl.when(lax.rem(i, 2) == 1)
                def _():
                    pltpu.make_async_copy(
                        msg_hbm.at[plane,
                                   pl.ds(sid * _S_EW + i * _S_BLK, _S_BLK)],
                        buf1, sem1).wait()
                    pltpu.sync_copy(buf1, acc.at[idx_v.at[j]], add=True)

        plsc.subcore_barrier()
        pltpu.sync_copy(acc.at[pl.ds(sid * _N_SUB, _N_SUB)],
                        dp_hbm.at[plane, pl.ds(sid * _N_SUB, _N_SUB)])
        plsc.subcore_barrier()


def _scatter_call(msg, dst_r, zeros_sub):
    k = pl.kernel(
        _scatter_body,
        out_type=jax.ShapeDtypeStruct((4, _NPAD, D), jnp.float32),
        mesh=_sc_mesh(),
        scratch_types=[
            pltpu.VMEM_SHARED((_NPAD, D), jnp.float32),
            pltpu.VMEM((_S_CHUNK, _S_BLK), jnp.int32),
            pltpu.VMEM((_S_BLK, D), jnp.float32),
            pltpu.VMEM((_S_BLK, D), jnp.float32),
            pltpu.SemaphoreType.DMA,
            pltpu.SemaphoreType.DMA,
        ],
    )
    return k(msg, dst_r, zeros_sub)


def kernel(node_states_scalar, node_states_vector, edge_states, edge_vectors,
           edge_norms, edge_index, Wf, bf, Ws1, bs1, Ws2, bs2, U, V,
           Wa1, ba1, Wa2, ba2):
    ns = node_states_scalar
    nsv_flat = node_states_vector.reshape(N, 3 * D)
    src = edge_index[:, 0]
    dst_r = jnp.concatenate(
        [edge_index[:, 1],
         jnp.full((_EPAD - E,), N, jnp.int32)]).reshape(_EPAD // _S_BLK,
                                                        _S_BLK)
    zeros_sub = jnp.zeros((_N_SUB, D), jnp.float32)

    bn = 1000
    so = pl.pallas_call(
        _a1_body,
        grid=(N // bn,),
        in_specs=[
            pl.BlockSpec((bn, D), lambda i: (i, 0)),
            pl.BlockSpec((D, D), lambda i: (0, 0)),
            pl.BlockSpec((1, D), lambda i: (0, 0)),
            pl.BlockSpec((D, 3 * D), lambda i: (0, 0)),
            pl.BlockSpec((1, 3 * D), lambda i: (0, 0)),
        ],
        out_specs=pl.BlockSpec((bn, 3 * D), lambda i: (i, 0)),
        out_shape=jax.ShapeDtypeStruct((N, 3 * D), jnp.float32),
    )(ns, Ws1, bs1.reshape(1, D), Ws2, bs2.reshape(1, 3 * D))

    be = 4000
    fw = pl.pallas_call(
        _a2_body,
        grid=(E // be,),
        in_specs=[
            pl.BlockSpec((be, DE), lambda i: (i, 0)),
            pl.BlockSpec((be, 1), lambda i: (i, 0)),
            pl.BlockSpec((DE, 3 * D), lambda i: (0, 0)),
            pl.BlockSpec((1, 3 * D), lambda i: (0, 0)),
        ],
        out_specs=pl.BlockSpec((be, 3 * D), lambda i: (i, 0)),
        out_shape=jax.ShapeDtypeStruct((E, 3 * D), jnp.float32),
    )(edge_states, edge_norms, Wf, bf.reshape(1, 3 * D))

    nsv_g = _gather_call(nsv_flat, src)
    so_g = _gather_call(so, src)

    bm = 2000
    msg = pl.pallas_call(
        _m_body,
        grid=(E // bm,),
        in_specs=[
            pl.BlockSpec((bm, 3 * D), lambda i: (i, 0)),
            pl.BlockSpec((bm, 3 * D), lambda i: (i, 0)),
            pl.BlockSpec((bm, 3 * D), lambda i: (i, 0)),
            pl.BlockSpec((bm, 3), lambda i: (i, 0)),
        ],
        out_specs=pl.BlockSpec((4, bm, D), lambda i: (0, i, 0)),
        out_shape=jax.ShapeDtypeStruct((4, _EPAD, D), jnp.float32),
    )(fw, so_g, nsv_g, edge_vectors)

    dp = _scatter_call(msg, dst_r, zeros_sub)

    outs, outv = pl.pallas_call(
        _b_body,
        grid=(N // bn,),
        in_specs=[
            pl.BlockSpec((bn, D), lambda i: (i, 0)),
            pl.BlockSpec((bn, 3 * D), lambda i: (i, 0)),
            pl.BlockSpec((4, bn, D), lambda i: (0, i, 0)),
            pl.BlockSpec((D, D), lambda i: (0, 0)),
            pl.BlockSpec((D, D), lambda i: (0, 0)),
            pl.BlockSpec((2 * D, D), lambda i: (0, 0)),
            pl.BlockSpec((1, D), lambda i: (0, 0)),
            pl.BlockSpec((D, 3 * D), lambda i: (0, 0)),
            pl.BlockSpec((1, 3 * D), lambda i: (0, 0)),
        ],
        out_specs=[
            pl.BlockSpec((bn, D), lambda i: (i, 0)),
            pl.BlockSpec((bn, 3 * D), lambda i: (i, 0)),
        ],
        out_shape=[
            jax.ShapeDtypeStruct((N, D), jnp.float32),
            jax.ShapeDtypeStruct((N, 3 * D), jnp.float32),
        ],
    )(ns, nsv_flat, dp, U, V, Wa1, ba1.reshape(1, D), Wa2,
      ba2.reshape(1, 3 * D))

    return outs, outv.reshape(N, 3, D)


# speedup vs baseline: 19.1595x; 1.0720x over previous
import functools

import jax
import jax.numpy as jnp
from jax import lax
from jax.experimental import pallas as pl
from jax.experimental.pallas import tpu as pltpu
from jax.experimental.pallas import tpu_sc as plsc

N = 10000
E = 320000
D = 128
DE = 16
CUTOFF = 5.0

NC = 2
NS = 16
NW = NC * NS


def _silu(x):
    return x * jax.nn.sigmoid(x)


def _pack2(a, b):
    bits_a = jax.lax.bitcast_convert_type(a, jnp.int32)
    bits_b = jax.lax.bitcast_convert_type(b, jnp.int32)
    lo = jax.lax.shift_right_logical(bits_a, 16) & jnp.int32(0xFFFF)
    return (bits_b & jnp.int32(-65536)) | lo


def _unpack_lo(w):
    return jax.lax.bitcast_convert_type(jax.lax.shift_left(w, 16),
                                        jnp.float32)


def _unpack_hi(w):
    return jax.lax.bitcast_convert_type(w & jnp.int32(-65536), jnp.float32)


def _a1_body(ns_ref, w1_ref, b1_ref, w2_ref, b2_ref, so_ref):
    h = _silu(jnp.dot(ns_ref[...], w1_ref[...],
                      preferred_element_type=jnp.float32) + b1_ref[...])
    so = (jnp.dot(h, w2_ref[...], preferred_element_type=jnp.float32)
          + b2_ref[...])
    so_ref[:, 0:D] = _pack2(so[:, 0:D], so[:, D:2 * D])
    so_ref[:, D:D + 64] = _pack2(so[:, 2 * D:2 * D + 64],
                                 so[:, 2 * D + 64:3 * D])
    so_ref[:, D + 64:2 * D] = jnp.zeros((so.shape[0], 64), jnp.int32)


def _a2_body(es_ref, norms_ref, wf_ref, bf_ref, fw_ref):
    fw = jnp.dot(es_ref[...], wf_ref[...],
                 preferred_element_type=jnp.float32) + bf_ref[...]
    r = norms_ref[...]
    cut = jnp.where(r < CUTOFF,
                    0.5 * (jnp.cos(jnp.pi * r / CUTOFF) + 1.0), 0.0)
    fw_ref[...] = (fw * cut).astype(jnp.bfloat16)


def _m_body(fw_ref, sog_ref, nsvg_ref, ev_ref, msg_ref):
    fw = fw_ref[...].astype(jnp.float32)
    x = sog_ref[...]
    so_a = _unpack_lo(x)
    so_b = _unpack_hi(x)
    gn = fw[:, 0:D] * so_a[:, 0:D]
    ge = fw[:, D:2 * D] * so_b[:, 0:D]
    msg_ref[0, :, 0:64] = fw[:, 2 * D:2 * D + 64] * so_a[:, D:D + 64]
    msg_ref[0, :, 64:D] = fw[:, 2 * D + 64:3 * D] * so_b[:, D:D + 64]
    y = nsvg_ref[...]
    nv_a = _unpack_lo(y)
    nv_b = _unpack_hi(y)
    ev = ev_ref[...]
    msg_ref[1] = nv_a[:, 0:D] * gn + ge * ev[:, 0:1]
    msg_ref[2] = nv_b[:, 0:D] * gn + ge * ev[:, 1:2]
    msg_ref[3, :, 0:64] = (nv_a[:, D:D + 64] * gn[:, 0:64]
                           + ge[:, 0:64] * ev[:, 2:3])
    msg_ref[3, :, 64:D] = (nv_b[:, D:D + 64] * gn[:, 64:D]
                           + ge[:, 64:D] * ev[:, 2:3])


def _b_body(ns_ref, nsv_ref, dp_ref, u_ref, v_ref, wa1_ref, ba1_ref,
            wa2_ref, ba2_ref, outs_ref, outv_ref):
    ns2 = ns_ref[...] + dp_ref[0]
    nv = [nsv_ref[:, k * D:(k + 1) * D] + dp_ref[1 + k] for k in range(3)]
    u = u_ref[...]
    v = v_ref[...]
    uv = [jnp.dot(nv[k], u, preferred_element_type=jnp.float32)
          for k in range(3)]
    vv = [jnp.dot(nv[k], v, preferred_element_type=jnp.float32)
          for k in range(3)]
    vv_sq = vv[0] * vv[0] + vv[1] * vv[1] + vv[2] * vv[2]
    inner = uv[0] * vv[0] + uv[1] * vv[1] + uv[2] * vv[2]
    wa1 = wa1_ref[...]
    h = (jnp.dot(ns2, wa1[0:D, :], preferred_element_type=jnp.float32)
         + jnp.dot(vv_sq, wa1[D:2 * D, :], preferred_element_type=jnp.float32)
         + ba1_ref[...])
    a = jnp.dot(_silu(h), wa2_ref[...],
                preferred_element_type=jnp.float32) + ba2_ref[...]
    outs_ref[...] = ns2 + a[:, 0:D] + a[:, D:2 * D] * inner
    for k in range(3):
        outv_ref[:, k * D:(k + 1) * D] = nv[k] + a[:, 2 * D:3 * D] * uv[k]


def _sc_mesh():
    return plsc.VectorSubcoreMesh(core_axis_name="c", subcore_axis_name="s",
                                  num_cores=NC, num_subcores=NS)

_G_EW = E // NW
_G_BLK = 80
_G_NB = _G_EW // _G_BLK

_S_BLK = 128
_EPAD = 2560 * _S_BLK
_S_EW = _EPAD // NS
_S_NB = _S_EW // _S_BLK
_S_CHUNK = 32
_NPAD = 10240
_N_SUB = _NPAD // NS


def _gather_rows_body(table_hbm, src_hbm, out_hbm, idx_v, buf0, buf1,
                      sem0, sem1):
    wid = lax.axis_index("s") * NC + lax.axis_index("c")
    base = wid * _G_EW
    pltpu.sync_copy(src_hbm.at[pl.ds(base, _G_EW)], idx_v)

    def gather(i, buf, sem):
        return pltpu.async_copy(
            table_hbm.at[idx_v.at[pl.ds(i * _G_BLK, _G_BLK)]], buf, sem)

    gather(0, buf0, sem0)

    @pl.loop(0, _G_NB)
    def _(i):
        nxt = i + 1

        @pl.when(nxt < _G_NB)
        def _():
            @pl.when(lax.rem(nxt, 2) == 0)
            def _():
                gather(nxt, buf0, sem0)

            @pl.when(lax.rem(nxt, 2) == 1)
            def _():
                gather(nxt, buf1, sem1)

        @pl.when(lax.rem(i, 2) == 0)
        def _():
            pltpu.make_async_copy(
                table_hbm.at[idx_v.at[pl.ds(i * _G_BLK, _G_BLK)]], buf0,
                sem0).wait()
            pltpu.sync_copy(buf0, out_hbm.at[pl.ds(base + i * _G_BLK, _G_BLK)])

        @pl.when(lax.rem(i, 2) == 1)
        def _():
            pltpu.make_async_copy(
                table_hbm.at[idx_v.at[pl.ds(i * _G_BLK, _G_BLK)]], buf1,
                sem1).wait()
            pltpu.sync_copy(buf1, out_hbm.at[pl.ds(base + i * _G_BLK, _G_BLK)])


def _gather_call(table, src):
    k = pl.kernel(
        _gather_rows_body,
        out_type=jax.ShapeDtypeStruct((E,) + table.shape[1:], table.dtype),
        mesh=_sc_mesh(),
        scratch_types=[
            pltpu.VMEM((_G_EW,), jnp.int32),
            pltpu.VMEM((_G_BLK,) + table.shape[1:], table.dtype),
            pltpu.VMEM((_G_BLK,) + table.shape[1:], table.dtype),
            pltpu.SemaphoreType.DMA,
            pltpu.SemaphoreType.DMA,
        ],
    )
    return k(table, src)


def _scatter_body(msg_hbm, dst_hbm, zeros_hbm, dp_hbm, acc, idx_v, buf0, buf1,
                  sem0, sem1):
    core = lax.axis_index("c")
    sid = lax.axis_index("s")

    for p_i in range(2):
        plane = 2 * core + p_i
        pltpu.sync_copy(zeros_hbm, acc.at[pl.ds(sid * _N_SUB, _N_SUB)])
        plsc.subcore_barrier()

        def load(i, buf, sem):
            return pltpu.async_copy(
                msg_hbm.at[plane, pl.ds(sid * _S_EW + i * _S_BLK, _S_BLK)],
                buf, sem)

        @pl.loop(0, _S_NB // _S_CHUNK)
        def _(chunk):
            pltpu.sync_copy(
                dst_hbm.at[pl.ds(sid * _S_NB + chunk * _S_CHUNK, _S_CHUNK)],
                idx_v)
            base = chunk * _S_CHUNK
            load(base, buf0, sem0)

            @pl.loop(0, _S_CHUNK)
            def _(j):
                i = base + j
                nxt = i + 1

                @pl.when(jnp.logical_and(nxt < _S_NB, j + 1 < _S_CHUNK))
                def _():
                    @pl.when(lax.rem(nxt, 2) == 0)
                    def _():
                        load(nxt, buf0, sem0)

                    @pl.when(lax.rem(nxt, 2) == 1)
                    def _():
                        load(nxt, buf1, sem1)

                @pl.when(lax.rem(i, 2) == 0)
                def _():
                    pltpu.make_async_copy(
                        msg_hbm.at[plane,
                                   pl.ds(sid * _S_EW + i * _S_BLK, _S_BLK)],
                        buf0, sem0).wait()
                    pltpu.sync_copy(buf0, acc.at[idx_v.at[j]], add=True)

                @pl.when(lax.rem(i, 2) == 1)
                def _():
                    pltpu.make_async_copy(
                        msg_hbm.at[plane,
                                   pl.ds(sid * _S_EW + i * _S_BLK, _S_BLK)],
                        buf1, sem1).wait()
                    pltpu.sync_copy(buf1, acc.at[idx_v.at[j]], add=True)

        plsc.subcore_barrier()
        pltpu.sync_copy(acc.at[pl.ds(sid * _N_SUB, _N_SUB)],
                        dp_hbm.at[plane, pl.ds(sid * _N_SUB, _N_SUB)])
        plsc.subcore_barrier()


def _scatter_call(msg, dst_r, zeros_sub):
    k = pl.kernel(
        _scatter_body,
        out_type=jax.ShapeDtypeStruct((4, _NPAD, D), jnp.float32),
        mesh=_sc_mesh(),
        scratch_types=[
            pltpu.VMEM_SHARED((_NPAD, D), jnp.float32),
            pltpu.VMEM((_S_CHUNK, _S_BLK), jnp.int32),
            pltpu.VMEM((_S_BLK, D), jnp.float32),
            pltpu.VMEM((_S_BLK, D), jnp.float32),
            pltpu.SemaphoreType.DMA,
            pltpu.SemaphoreType.DMA,
        ],
    )
    return k(msg, dst_r, zeros_sub)


def kernel(node_states_scalar, node_states_vector, edge_states, edge_vectors,
           edge_norms, edge_index, Wf, bf, Ws1, bs1, Ws2, bs2, U, V,
           Wa1, ba1, Wa2, ba2):
    ns = node_states_scalar
    nsv_flat = node_states_vector.reshape(N, 3 * D)
    nsv_p = jnp.concatenate(
        [_pack2(node_states_vector[:, 0, :], node_states_vector[:, 1, :]),
         _pack2(node_states_vector[:, 2, 0:64],
                node_states_vector[:, 2, 64:D]),
         jnp.zeros((N, 64), jnp.int32)],
        axis=1)
    src = edge_index[:, 0]
    dst_r = jnp.concatenate(
        [edge_index[:, 1],
         jnp.full((_EPAD - E,), N, jnp.int32)]).reshape(_EPAD // _S_BLK,
                                                        _S_BLK)
    zeros_sub = jnp.zeros((_N_SUB, D), jnp.float32)

    bn = 1000
    so = pl.pallas_call(
        _a1_body,
        grid=(N // bn,),
        in_specs=[
            pl.BlockSpec((bn, D), lambda i: (i, 0)),
            pl.BlockSpec((D, D), lambda i: (0, 0)),
            pl.BlockSpec((1, D), lambda i: (0, 0)),
            pl.BlockSpec((D, 3 * D), lambda i: (0, 0)),
            pl.BlockSpec((1, 3 * D), lambda i: (0, 0)),
        ],
        out_specs=pl.BlockSpec((bn, 2 * D), lambda i: (i, 0)),
        out_shape=jax.ShapeDtypeStruct((N, 2 * D), jnp.int32),
    )(ns, Ws1, bs1.reshape(1, D), Ws2, bs2.reshape(1, 3 * D))

    be = 4000
    fw = pl.pallas_call(
        _a2_body,
        grid=(E // be,),
        in_specs=[
            pl.BlockSpec((be, DE), lambda i: (i, 0)),
            pl.BlockSpec((be, 1), lambda i: (i, 0)),
            pl.BlockSpec((DE, 3 * D), lambda i: (0, 0)),
            pl.BlockSpec((1, 3 * D), lambda i: (0, 0)),
        ],
        out_specs=pl.BlockSpec((be, 3 * D), lambda i: (i, 0)),
        out_shape=jax.ShapeDtypeStruct((E, 3 * D), jnp.bfloat16),
    )(edge_states, edge_norms, Wf, bf.reshape(1, 3 * D))

    nsv_g = _gather_call(nsv_p, src)
    so_g = _gather_call(so, src)

    bm = 2000
    msg = pl.pallas_call(
        _m_body,
        grid=(E // bm,),
        in_specs=[
            pl.BlockSpec((bm, 3 * D), lambda i: (i, 0)),
            pl.BlockSpec((bm, 2 * D), lambda i: (i, 0)),
            pl.BlockSpec((bm, 2 * D), lambda i: (i, 0)),
            pl.BlockSpec((bm, 3), lambda i: (i, 0)),
        ],
        out_specs=pl.BlockSpec((4, bm, D), lambda i: (0, i, 0)),
        out_shape=jax.ShapeDtypeStruct((4, _EPAD, D), jnp.float32),
    )(fw, so_g, nsv_g, edge_vectors)

    dp = _scatter_call(msg, dst_r, zeros_sub)

    outs, outv = pl.pallas_call(
        _b_body,
        grid=(N // bn,),
        in_specs=[
            pl.BlockSpec((bn, D), lambda i: (i, 0)),
            pl.BlockSpec((bn, 3 * D), lambda i: (i, 0)),
            pl.BlockSpec((4, bn, D), lambda i: (0, i, 0)),
            pl.BlockSpec((D, D), lambda i: (0, 0)),
            pl.BlockSpec((D, D), lambda i: (0, 0)),
            pl.BlockSpec((2 * D, D), lambda i: (0, 0)),
            pl.BlockSpec((1, D), lambda i: (0, 0)),
            pl.BlockSpec((D, 3 * D), lambda i: (0, 0)),
            pl.BlockSpec((1, 3 * D), lambda i: (0, 0)),
        ],
        out_specs=[
            pl.BlockSpec((bn, D), lambda i: (i, 0)),
            pl.BlockSpec((bn, 3 * D), lambda i: (i, 0)),
        ],
        out_shape=[
            jax.ShapeDtypeStruct((N, D), jnp.float32),
            jax.ShapeDtypeStruct((N, 3 * D), jnp.float32),
        ],
    )(ns, nsv_flat, dp, U, V, Wa1, ba1.reshape(1, D), Wa2,
      ba2.reshape(1, 3 * D))

    return outs, outv.reshape(N, 3, D)


# speedup vs baseline: 19.2641x; 1.0055x over previous
import functools

import jax
import jax.numpy as jnp
from jax import lax
from jax.experimental import pallas as pl
from jax.experimental.pallas import tpu as pltpu
from jax.experimental.pallas import tpu_sc as plsc

N = 10000
E = 320000
D = 128
DE = 16
CUTOFF = 5.0

NC = 2
NS = 16
NW = NC * NS


def _silu(x):
    return x * jax.nn.sigmoid(x)


def _pack2(a, b):
    bits_a = jax.lax.bitcast_convert_type(a, jnp.int32)
    bits_b = jax.lax.bitcast_convert_type(b, jnp.int32)
    lo = jax.lax.shift_right_logical(bits_a, 16) & jnp.int32(0xFFFF)
    return (bits_b & jnp.int32(-65536)) | lo


def _unpack_lo(w):
    return jax.lax.bitcast_convert_type(jax.lax.shift_left(w, 16),
                                        jnp.float32)


def _unpack_hi(w):
    return jax.lax.bitcast_convert_type(w & jnp.int32(-65536), jnp.float32)


def _a1_body(ns_ref, w1_ref, b1_ref, w2_ref, b2_ref, so_ref):
    h = _silu(jnp.dot(ns_ref[...], w1_ref[...],
                      preferred_element_type=jnp.float32) + b1_ref[...])
    so = (jnp.dot(h, w2_ref[...], preferred_element_type=jnp.float32)
          + b2_ref[...])
    so_ref[:, 0:D] = _pack2(so[:, 0:D], so[:, D:2 * D])
    so_ref[:, D:D + 64] = _pack2(so[:, 2 * D:2 * D + 64],
                                 so[:, 2 * D + 64:3 * D])
    so_ref[:, D + 64:2 * D] = jnp.zeros((so.shape[0], 64), jnp.int32)


def _a2_body(es_ref, norms_ref, wf_ref, bf_ref, fw_ref):
    fw = jnp.dot(es_ref[...], wf_ref[...],
                 preferred_element_type=jnp.float32) + bf_ref[...]
    r = norms_ref[...]
    cut = jnp.where(r < CUTOFF,
                    0.5 * (jnp.cos(jnp.pi * r / CUTOFF) + 1.0), 0.0)
    fw_ref[...] = (fw * cut).astype(jnp.bfloat16)


def _m_body(fw_ref, sog_ref, nsvg_ref, ev_ref, msg_ref):
    fw = fw_ref[...].astype(jnp.float32)
    x = sog_ref[...]
    so_a = _unpack_lo(x)
    so_b = _unpack_hi(x)
    gn = fw[:, 0:D] * so_a[:, 0:D]
    ge = fw[:, D:2 * D] * so_b[:, 0:D]
    msg_ref[0, :, 0:64] = fw[:, 2 * D:2 * D + 64] * so_a[:, D:D + 64]
    msg_ref[0, :, 64:D] = fw[:, 2 * D + 64:3 * D] * so_b[:, D:D + 64]
    y = nsvg_ref[...]
    nv_a = _unpack_lo(y)
    nv_b = _unpack_hi(y)
    ev = ev_ref[...]
    msg_ref[1] = nv_a[:, 0:D] * gn + ge * ev[:, 0:1]
    msg_ref[2] = nv_b[:, 0:D] * gn + ge * ev[:, 1:2]
    msg_ref[3, :, 0:64] = (nv_a[:, D:D + 64] * gn[:, 0:64]
                           + ge[:, 0:64] * ev[:, 2:3])
    msg_ref[3, :, 64:D] = (nv_b[:, D:D + 64] * gn[:, 64:D]
                           + ge[:, 64:D] * ev[:, 2:3])


def _b_body(ns_ref, nsv_ref, dp_ref, u_ref, v_ref, wa1_ref, ba1_ref,
            wa2_ref, ba2_ref, outs_ref, outv_ref):
    ns2 = ns_ref[...] + dp_ref[0]
    nv = [nsv_ref[:, k * D:(k + 1) * D] + dp_ref[1 + k] for k in range(3)]
    u = u_ref[...]
    v = v_ref[...]
    uv = [jnp.dot(nv[k], u, preferred_element_type=jnp.float32)
          for k in range(3)]
    vv = [jnp.dot(nv[k], v, preferred_element_type=jnp.float32)
          for k in range(3)]
    vv_sq = vv[0] * vv[0] + vv[1] * vv[1] + vv[2] * vv[2]
    inner = uv[0] * vv[0] + uv[1] * vv[1] + uv[2] * vv[2]
    wa1 = wa1_ref[...]
    h = (jnp.dot(ns2, wa1[0:D, :], preferred_element_type=jnp.float32)
         + jnp.dot(vv_sq, wa1[D:2 * D, :], preferred_element_type=jnp.float32)
         + ba1_ref[...])
    a = jnp.dot(_silu(h), wa2_ref[...],
                preferred_element_type=jnp.float32) + ba2_ref[...]
    outs_ref[...] = ns2 + a[:, 0:D] + a[:, D:2 * D] * inner
    for k in range(3):
        outv_ref[:, k * D:(k + 1) * D] = nv[k] + a[:, 2 * D:3 * D] * uv[k]


def _sc_mesh():
    return plsc.VectorSubcoreMesh(core_axis_name="c", subcore_axis_name="s",
                                  num_cores=NC, num_subcores=NS)

_G_EW = E // NW
_G_BLK = 80
_G_NB = _G_EW // _G_BLK

_S_BLK = 128
_EPAD = 2560 * _S_BLK
_S_EW = _EPAD // NS
_S_NB = _S_EW // _S_BLK
_S_CHUNK = 32
_NPAD = 10240
_N_SUB = _NPAD // NS


def _gather2_body(ta_hbm, tb_hbm, src_hbm, oa_hbm, ob_hbm, idx_v,
                  bufa0, bufa1, bufb0, bufb1, sa0, sa1, sb0, sb1):
    wid = lax.axis_index("s") * NC + lax.axis_index("c")
    base = wid * _G_EW
    pltpu.sync_copy(src_hbm.at[pl.ds(base, _G_EW)], idx_v)
    bufs = ((bufa0, bufb0, sa0, sb0), (bufa1, bufb1, sa1, sb1))

    def gather(i, p):
        ba, bb, sa, sb = bufs[p]
        pltpu.async_copy(ta_hbm.at[idx_v.at[pl.ds(i * _G_BLK, _G_BLK)]],
                         ba, sa)
        pltpu.async_copy(tb_hbm.at[idx_v.at[pl.ds(i * _G_BLK, _G_BLK)]],
                         bb, sb)

    def wait_pair(p):
        ba, bb, sa, sb = bufs[p]
        pltpu.make_async_copy(ta_hbm.at[idx_v.at[pl.ds(0, _G_BLK)]], ba,
                              sa).wait()
        pltpu.make_async_copy(tb_hbm.at[idx_v.at[pl.ds(0, _G_BLK)]], bb,
                              sb).wait()

    def writeback(i, p):
        ba, bb, sa, sb = bufs[p]
        pltpu.async_copy(ba, oa_hbm.at[pl.ds(base + i * _G_BLK, _G_BLK)], sa)
        pltpu.async_copy(bb, ob_hbm.at[pl.ds(base + i * _G_BLK, _G_BLK)], sb)

    gather(0, 0)

    @pl.loop(0, _G_NB)
    def _(i):
        nxt = i + 1
        par_i = lax.rem(i, 2)
        par_n = lax.rem(nxt, 2)

        @pl.when(nxt < _G_NB)
        def _():
            @pl.when(par_n == 0)
            def _():
                @pl.when(nxt >= 2)
                def _():
                    wait_pair(0)
                gather(nxt, 0)

            @pl.when(par_n == 1)
            def _():
                @pl.when(nxt >= 2)
                def _():
                    wait_pair(1)
                gather(nxt, 1)

        @pl.when(par_i == 0)
        def _():
            wait_pair(0)
            writeback(i, 0)

        @pl.when(par_i == 1)
        def _():
            wait_pair(1)
            writeback(i, 1)

    wait_pair((_G_NB - 1) % 2)
    wait_pair((_G_NB - 2) % 2)


def _gather2_call(ta, tb, src):
    def buft(t):
        return pltpu.VMEM((_G_BLK,) + t.shape[1:], t.dtype)

    k = pl.kernel(
        _gather2_body,
        out_type=(jax.ShapeDtypeStruct((E,) + ta.shape[1:], ta.dtype),
                  jax.ShapeDtypeStruct((E,) + tb.shape[1:], tb.dtype)),
        mesh=_sc_mesh(),
        scratch_types=[
            pltpu.VMEM((_G_EW,), jnp.int32),
            buft(ta), buft(ta), buft(tb), buft(tb),
            pltpu.SemaphoreType.DMA,
            pltpu.SemaphoreType.DMA,
            pltpu.SemaphoreType.DMA,
            pltpu.SemaphoreType.DMA,
        ],
    )
    return k(ta, tb, src)


def _scatter_body(msg_hbm, dst_hbm, zeros_hbm, dp_hbm, acc, idx_v, buf0, buf1,
                  sem0, sem1):
    core = lax.axis_index("c")
    sid = lax.axis_index("s")

    for p_i in range(2):
        plane = 2 * core + p_i
        pltpu.sync_copy(zeros_hbm, acc.at[pl.ds(sid * _N_SUB, _N_SUB)])
        plsc.subcore_barrier()

        def load(i, buf, sem):
            return pltpu.async_copy(
                msg_hbm.at[plane, pl.ds(sid * _S_EW + i * _S_BLK, _S_BLK)],
                buf, sem)

        bufs = ((buf0, sem0), (buf1, sem1))

        def wait_sem(p):
            b, s = bufs[p]
            pltpu.make_async_copy(
                msg_hbm.at[plane, pl.ds(sid * _S_EW, _S_BLK)], b, s).wait()

        def scatter(j, p):
            b, s = bufs[p]
            pltpu.async_copy(b, acc.at[idx_v.at[j]], s, add=True)

        @pl.loop(0, _S_NB // _S_CHUNK)
        def _(chunk):
            @pl.when(chunk > 0)
            def _():
                wait_sem(0)
                wait_sem(1)
            pltpu.sync_copy(
                dst_hbm.at[pl.ds(sid * _S_NB + chunk * _S_CHUNK, _S_CHUNK)],
                idx_v)
            base = chunk * _S_CHUNK
            load(base, buf0, sem0)

            @pl.loop(0, _S_CHUNK)
            def _(j):
                i = base + j
                nxt = i + 1

                @pl.when(j + 1 < _S_CHUNK)
                def _():
                    @pl.when(lax.rem(nxt, 2) == 0)
                    def _():
                        @pl.when(j + 1 >= 2)
                        def _():
                            wait_sem(0)
                        load(nxt, buf0, sem0)

                    @pl.when(lax.rem(nxt, 2) == 1)
                    def _():
                        @pl.when(j + 1 >= 2)
                        def _():
                            wait_sem(1)
                        load(nxt, buf1, sem1)

                @pl.when(lax.rem(i, 2) == 0)
                def _():
                    wait_sem(0)
                    scatter(j, 0)

                @pl.when(lax.rem(i, 2) == 1)
                def _():
                    wait_sem(1)
                    scatter(j, 1)

        wait_sem(0)
        wait_sem(1)
        plsc.subcore_barrier()
        pltpu.sync_copy(acc.at[pl.ds(sid * _N_SUB, _N_SUB)],
                        dp_hbm.at[plane, pl.ds(sid * _N_SUB, _N_SUB)])
        plsc.subcore_barrier()


def _scatter_call(msg, dst_r, zeros_sub):
    k = pl.kernel(
        _scatter_body,
        out_type=jax.ShapeDtypeStruct((4, _NPAD, D), jnp.float32),
        mesh=_sc_mesh(),
        scratch_types=[
            pltpu.VMEM_SHARED((_NPAD, D), jnp.float32),
            pltpu.VMEM((_S_CHUNK, _S_BLK), jnp.int32),
            pltpu.VMEM((_S_BLK, D), jnp.float32),
            pltpu.VMEM((_S_BLK, D), jnp.float32),
            pltpu.SemaphoreType.DMA,
            pltpu.SemaphoreType.DMA,
        ],
    )
    return k(msg, dst_r, zeros_sub)


def kernel(node_states_scalar, node_states_vector, edge_states, edge_vectors,
           edge_norms, edge_index, Wf, bf, Ws1, bs1, Ws2, bs2, U, V,
           Wa1, ba1, Wa2, ba2):
    ns = node_states_scalar
    nsv_flat = node_states_vector.reshape(N, 3 * D)
    nsv_p = jnp.concatenate(
        [_pack2(node_states_vector[:, 0, :], node_states_vector[:, 1, :]),
         _pack2(node_states_vector[:, 2, 0:64],
                node_states_vector[:, 2, 64:D]),
         jnp.zeros((N, 64), jnp.int32)],
        axis=1)
    src = edge_index[:, 0]
    dst_r = jnp.concatenate(
        [edge_index[:, 1],
         jnp.full((_EPAD - E,), N, jnp.int32)]).reshape(_EPAD // _S_BLK,
                                                        _S_BLK)
    zeros_sub = jnp.zeros((_N_SUB, D), jnp.float32)

    bn = 1000
    so = pl.pallas_call(
        _a1_body,
        grid=(N // bn,),
        in_specs=[
            pl.BlockSpec((bn, D), lambda i: (i, 0)),
            pl.BlockSpec((D, D), lambda i: (0, 0)),
            pl.BlockSpec((1, D), lambda i: (0, 0)),
            pl.BlockSpec((D, 3 * D), lambda i: (0, 0)),
            pl.BlockSpec((1, 3 * D), lambda i: (0, 0)),
        ],
        out_specs=pl.BlockSpec((bn, 2 * D), lambda i: (i, 0)),
        out_shape=jax.ShapeDtypeStruct((N, 2 * D), jnp.int32),
    )(ns, Ws1, bs1.reshape(1, D), Ws2, bs2.reshape(1, 3 * D))

    be = 4000
    fw = pl.pallas_call(
        _a2_body,
        grid=(E // be,),
        in_specs=[
            pl.BlockSpec((be, DE), lambda i: (i, 0)),
            pl.BlockSpec((be, 1), lambda i: (i, 0)),
            pl.BlockSpec((DE, 3 * D), lambda i: (0, 0)),
            pl.BlockSpec((1, 3 * D), lambda i: (0, 0)),
        ],
        out_specs=pl.BlockSpec((be, 3 * D), lambda i: (i, 0)),
        out_shape=jax.ShapeDtypeStruct((E, 3 * D), jnp.bfloat16),
    )(edge_states, edge_norms, Wf, bf.reshape(1, 3 * D))

    nsv_g, so_g = _gather2_call(nsv_p, so, src)

    bm = 2000
    msg = pl.pallas_call(
        _m_body,
        grid=(E // bm,),
        in_specs=[
            pl.BlockSpec((bm, 3 * D), lambda i: (i, 0)),
            pl.BlockSpec((bm, 2 * D), lambda i: (i, 0)),
            pl.BlockSpec((bm, 2 * D), lambda i: (i, 0)),
            pl.BlockSpec((bm, 3), lambda i: (i, 0)),
        ],
        out_specs=pl.BlockSpec((4, bm, D), lambda i: (0, i, 0)),
        out_shape=jax.ShapeDtypeStruct((4, _EPAD, D), jnp.float32),
    )(fw, so_g, nsv_g, edge_vectors)

    dp = _scatter_call(msg, dst_r, zeros_sub)

    outs, outv = pl.pallas_call(
        _b_body,
        grid=(N // bn,),
        in_specs=[
            pl.BlockSpec((bn, D), lambda i: (i, 0)),
            pl.BlockSpec((bn, 3 * D), lambda i: (i, 0)),
            pl.BlockSpec((4, bn, D), lambda i: (0, i, 0)),
            pl.BlockSpec((D, D), lambda i: (0, 0)),
            pl.BlockSpec((D, D), lambda i: (0, 0)),
            pl.BlockSpec((2 * D, D), lambda i: (0, 0)),
            pl.BlockSpec((1, D), lambda i: (0, 0)),
            pl.BlockSpec((D, 3 * D), lambda i: (0, 0)),
            pl.BlockSpec((1, 3 * D), lambda i: (0, 0)),
        ],
        out_specs=[
            pl.BlockSpec((bn, D), lambda i: (i, 0)),
            pl.BlockSpec((bn, 3 * D), lambda i: (i, 0)),
        ],
        out_shape=[
            jax.ShapeDtypeStruct((N, D), jnp.float32),
            jax.ShapeDtypeStruct((N, 3 * D), jnp.float32),
        ],
    )(ns, nsv_flat, dp, U, V, Wa1, ba1.reshape(1, D), Wa2,
      ba2.reshape(1, 3 * D))

    return outs, outv.reshape(N, 3, D)


# speedup vs baseline: 20.6336x; 1.0711x over previous
import functools

import jax
import jax.numpy as jnp
from jax import lax
from jax.experimental import pallas as pl
from jax.experimental.pallas import tpu as pltpu
from jax.experimental.pallas import tpu_sc as plsc

N = 10000
E = 320000
D = 128
DE = 16
CUTOFF = 5.0

NC = 2
NS = 16
NW = NC * NS


def _silu(x):
    return x * jax.nn.sigmoid(x)


def _pack2(a, b):
    bits_a = jax.lax.bitcast_convert_type(a, jnp.int32)
    bits_b = jax.lax.bitcast_convert_type(b, jnp.int32)
    lo = jax.lax.shift_right_logical(bits_a, 16) & jnp.int32(0xFFFF)
    return (bits_b & jnp.int32(-65536)) | lo


def _unpack_lo(w):
    return jax.lax.bitcast_convert_type(jax.lax.shift_left(w, 16),
                                        jnp.float32)


def _unpack_hi(w):
    return jax.lax.bitcast_convert_type(w & jnp.int32(-65536), jnp.float32)


def _a1_body(ns_ref, w1_ref, b1_ref, w2_ref, b2_ref, so_ref):
    h = _silu(jnp.dot(ns_ref[...], w1_ref[...],
                      preferred_element_type=jnp.float32) + b1_ref[...])
    so = (jnp.dot(h, w2_ref[...], preferred_element_type=jnp.float32)
          + b2_ref[...])
    so_ref[:, 0:D] = _pack2(so[:, 0:D], so[:, D:2 * D])
    so_ref[:, D:D + 64] = _pack2(so[:, 2 * D:2 * D + 64],
                                 so[:, 2 * D + 64:3 * D])
    so_ref[:, D + 64:2 * D] = jnp.zeros((so.shape[0], 64), jnp.int32)


def _a2_body(es_ref, norms_ref, wf_ref, bf_ref, fw_ref):
    fw = jnp.dot(es_ref[...], wf_ref[...],
                 preferred_element_type=jnp.float32) + bf_ref[...]
    r = norms_ref[...]
    cut = jnp.where(r < CUTOFF,
                    0.5 * (jnp.cos(jnp.pi * r / CUTOFF) + 1.0), 0.0)
    fw_ref[...] = (fw * cut).astype(jnp.bfloat16)


def _m_body(es_ref, norms_ref, wf_ref, bf_ref, sog_ref, nsvg_ref, ev_ref,
            msg_ref):
    fwl = jnp.dot(es_ref[...], wf_ref[...],
                  preferred_element_type=jnp.float32) + bf_ref[...]
    r = norms_ref[...]
    cut = jnp.where(r < CUTOFF,
                    0.5 * (jnp.cos(jnp.pi * r / CUTOFF) + 1.0), 0.0)
    fw = fwl * cut
    x = sog_ref[...]
    so_a = _unpack_lo(x)
    so_b = _unpack_hi(x)
    gn = fw[:, 0:D] * so_a[:, 0:D]
    ge = fw[:, D:2 * D] * so_b[:, 0:D]
    msg_ref[0, :, 0:64] = fw[:, 2 * D:2 * D + 64] * so_a[:, D:D + 64]
    msg_ref[0, :, 64:D] = fw[:, 2 * D + 64:3 * D] * so_b[:, D:D + 64]
    y = nsvg_ref[...]
    nv_a = _unpack_lo(y)
    nv_b = _unpack_hi(y)
    ev = ev_ref[...]
    msg_ref[1] = nv_a[:, 0:D] * gn + ge * ev[:, 0:1]
    msg_ref[2] = nv_b[:, 0:D] * gn + ge * ev[:, 1:2]
    msg_ref[3, :, 0:64] = (nv_a[:, D:D + 64] * gn[:, 0:64]
                           + ge[:, 0:64] * ev[:, 2:3])
    msg_ref[3, :, 64:D] = (nv_b[:, D:D + 64] * gn[:, 64:D]
                           + ge[:, 64:D] * ev[:, 2:3])


def _b_body(ns_ref, nsv_ref, dp_ref, u_ref, v_ref, wa1_ref, ba1_ref,
            wa2_ref, ba2_ref, outs_ref, outv_ref):
    ns2 = ns_ref[...] + dp_ref[0]
    nv = [nsv_ref[:, k * D:(k + 1) * D] + dp_ref[1 + k] for k in range(3)]
    u = u_ref[...]
    v = v_ref[...]
    uv = [jnp.dot(nv[k], u, preferred_element_type=jnp.float32)
          for k in range(3)]
    vv = [jnp.dot(nv[k], v, preferred_element_type=jnp.float32)
          for k in range(3)]
    vv_sq = vv[0] * vv[0] + vv[1] * vv[1] + vv[2] * vv[2]
    inner = uv[0] * vv[0] + uv[1] * vv[1] + uv[2] * vv[2]
    wa1 = wa1_ref[...]
    h = (jnp.dot(ns2, wa1[0:D, :], preferred_element_type=jnp.float32)
         + jnp.dot(vv_sq, wa1[D:2 * D, :], preferred_element_type=jnp.float32)
         + ba1_ref[...])
    a = jnp.dot(_silu(h), wa2_ref[...],
                preferred_element_type=jnp.float32) + ba2_ref[...]
    outs_ref[...] = ns2 + a[:, 0:D] + a[:, D:2 * D] * inner
    for k in range(3):
        outv_ref[:, k * D:(k + 1) * D] = nv[k] + a[:, 2 * D:3 * D] * uv[k]


def _sc_mesh():
    return plsc.VectorSubcoreMesh(core_axis_name="c", subcore_axis_name="s",
                                  num_cores=NC, num_subcores=NS)

_G_EW = E // NW
_G_BLK = 80
_G_NB = _G_EW // _G_BLK

_S_BLK = 128
_EPAD = 2560 * _S_BLK
_S_EW = _EPAD // NS
_S_NB = _S_EW // _S_BLK
_S_CHUNK = 32
_NPAD = 10240
_N_SUB = _NPAD // NS


def _gather2_body(ta_hbm, tb_hbm, src_hbm, oa_hbm, ob_hbm, idx_v,
                  bufa0, bufa1, bufb0, bufb1, sa0, sa1, sb0, sb1):
    wid = lax.axis_index("s") * NC + lax.axis_index("c")
    base = wid * _G_EW
    pltpu.sync_copy(src_hbm.at[pl.ds(base, _G_EW)], idx_v)
    bufs = ((bufa0, bufb0, sa0, sb0), (bufa1, bufb1, sa1, sb1))

    def gather(i, p):
        ba, bb, sa, sb = bufs[p]
        pltpu.async_copy(ta_hbm.at[idx_v.at[pl.ds(i * _G_BLK, _G_BLK)]],
                         ba, sa)
        pltpu.async_copy(tb_hbm.at[idx_v.at[pl.ds(i * _G_BLK, _G_BLK)]],
                         bb, sb)

    def wait_pair(p):
        ba, bb, sa, sb = bufs[p]
        pltpu.make_async_copy(ta_hbm.at[idx_v.at[pl.ds(0, _G_BLK)]], ba,
                              sa).wait()
        pltpu.make_async_copy(tb_hbm.at[idx_v.at[pl.ds(0, _G_BLK)]], bb,
                              sb).wait()

    def writeback(i, p):
        ba, bb, sa, sb = bufs[p]
        pltpu.async_copy(ba, oa_hbm.at[pl.ds(base + i * _G_BLK, _G_BLK)], sa)
        pltpu.async_copy(bb, ob_hbm.at[pl.ds(base + i * _G_BLK, _G_BLK)], sb)

    gather(0, 0)

    @pl.loop(0, _G_NB)
    def _(i):
        nxt = i + 1
        par_i = lax.rem(i, 2)
        par_n = lax.rem(nxt, 2)

        @pl.when(nxt < _G_NB)
        def _():
            @pl.when(par_n == 0)
            def _():
                @pl.when(nxt >= 2)
                def _():
                    wait_pair(0)
                gather(nxt, 0)

            @pl.when(par_n == 1)
            def _():
                @pl.when(nxt >= 2)
                def _():
                    wait_pair(1)
                gather(nxt, 1)

        @pl.when(par_i == 0)
        def _():
            wait_pair(0)
            writeback(i, 0)

        @pl.when(par_i == 1)
        def _():
            wait_pair(1)
            writeback(i, 1)

    wait_pair((_G_NB - 1) % 2)
    wait_pair((_G_NB - 2) % 2)


def _gather2_call(ta, tb, src):
    def buft(t):
        return pltpu.VMEM((_G_BLK,) + t.shape[1:], t.dtype)

    k = pl.kernel(
        _gather2_body,
        out_type=(jax.ShapeDtypeStruct((E,) + ta.shape[1:], ta.dtype),
                  jax.ShapeDtypeStruct((E,) + tb.shape[1:], tb.dtype)),
        mesh=_sc_mesh(),
        scratch_types=[
            pltpu.VMEM((_G_EW,), jnp.int32),
            buft(ta), buft(ta), buft(tb), buft(tb),
            pltpu.SemaphoreType.DMA,
            pltpu.SemaphoreType.DMA,
            pltpu.SemaphoreType.DMA,
            pltpu.SemaphoreType.DMA,
        ],
    )
    return k(ta, tb, src)


def _scatter_body(msg_hbm, dst_hbm, zeros_hbm, dp_hbm, acc, idx_v, buf0, buf1,
                  sem0, sem1):
    core = lax.axis_index("c")
    sid = lax.axis_index("s")

    for p_i in range(2):
        plane = 2 * core + p_i
        pltpu.sync_copy(zeros_hbm, acc.at[pl.ds(sid * _N_SUB, _N_SUB)])
        plsc.subcore_barrier()

        def load(i, buf, sem):
            return pltpu.async_copy(
                msg_hbm.at[plane, pl.ds(sid * _S_EW + i * _S_BLK, _S_BLK)],
                buf, sem)

        bufs = ((buf0, sem0), (buf1, sem1))

        def wait_sem(p):
            b, s = bufs[p]
            pltpu.make_async_copy(
                msg_hbm.at[plane, pl.ds(sid * _S_EW, _S_BLK)], b, s).wait()

        def scatter(j, p):
            b, s = bufs[p]
            pltpu.async_copy(b, acc.at[idx_v.at[j]], s, add=True)

        @pl.loop(0, _S_NB // _S_CHUNK)
        def _(chunk):
            @pl.when(chunk > 0)
            def _():
                wait_sem(0)
                wait_sem(1)
            pltpu.sync_copy(
                dst_hbm.at[pl.ds(sid * _S_NB + chunk * _S_CHUNK, _S_CHUNK)],
                idx_v)
            base = chunk * _S_CHUNK
            load(base, buf0, sem0)

            @pl.loop(0, _S_CHUNK)
            def _(j):
                i = base + j
                nxt = i + 1

                @pl.when(j + 1 < _S_CHUNK)
                def _():
                    @pl.when(lax.rem(nxt, 2) == 0)
                    def _():
                        @pl.when(j + 1 >= 2)
                        def _():
                            wait_sem(0)
                        load(nxt, buf0, sem0)

                    @pl.when(lax.rem(nxt, 2) == 1)
                    def _():
                        @pl.when(j + 1 >= 2)
                        def _():
                            wait_sem(1)
                        load(nxt, buf1, sem1)

                @pl.when(lax.rem(i, 2) == 0)
                def _():
                    wait_sem(0)
                    scatter(j, 0)

                @pl.when(lax.rem(i, 2) == 1)
                def _():
                    wait_sem(1)
                    scatter(j, 1)

        wait_sem(0)
        wait_sem(1)
        plsc.subcore_barrier()
        pltpu.sync_copy(acc.at[pl.ds(sid * _N_SUB, _N_SUB)],
                        dp_hbm.at[plane, pl.ds(sid * _N_SUB, _N_SUB)])
        plsc.subcore_barrier()


def _scatter_call(msg, dst_r, zeros_sub):
    k = pl.kernel(
        _scatter_body,
        out_type=jax.ShapeDtypeStruct((4, _NPAD, D), jnp.float32),
        mesh=_sc_mesh(),
        scratch_types=[
            pltpu.VMEM_SHARED((_NPAD, D), jnp.float32),
            pltpu.VMEM((_S_CHUNK, _S_BLK), jnp.int32),
            pltpu.VMEM((_S_BLK, D), jnp.float32),
            pltpu.VMEM((_S_BLK, D), jnp.float32),
            pltpu.SemaphoreType.DMA,
            pltpu.SemaphoreType.DMA,
        ],
    )
    return k(msg, dst_r, zeros_sub)


def kernel(node_states_scalar, node_states_vector, edge_states, edge_vectors,
           edge_norms, edge_index, Wf, bf, Ws1, bs1, Ws2, bs2, U, V,
           Wa1, ba1, Wa2, ba2):
    ns = node_states_scalar
    nsv_flat = node_states_vector.reshape(N, 3 * D)
    nsv_p = jnp.concatenate(
        [_pack2(node_states_vector[:, 0, :], node_states_vector[:, 1, :]),
         _pack2(node_states_vector[:, 2, 0:64],
                node_states_vector[:, 2, 64:D]),
         jnp.zeros((N, 64), jnp.int32)],
        axis=1)
    src = edge_index[:, 0]
    dst_r = jnp.concatenate(
        [edge_index[:, 1],
         jnp.full((_EPAD - E,), N, jnp.int32)]).reshape(_EPAD // _S_BLK,
                                                        _S_BLK)
    zeros_sub = jnp.zeros((_N_SUB, D), jnp.float32)

    bn = 1000
    so = pl.pallas_call(
        _a1_body,
        grid=(N // bn,),
        in_specs=[
            pl.BlockSpec((bn, D), lambda i: (i, 0)),
            pl.BlockSpec((D, D), lambda i: (0, 0)),
            pl.BlockSpec((1, D), lambda i: (0, 0)),
            pl.BlockSpec((D, 3 * D), lambda i: (0, 0)),
            pl.BlockSpec((1, 3 * D), lambda i: (0, 0)),
        ],
        out_specs=pl.BlockSpec((bn, 2 * D), lambda i: (i, 0)),
        out_shape=jax.ShapeDtypeStruct((N, 2 * D), jnp.int32),
    )(ns, Ws1, bs1.reshape(1, D), Ws2, bs2.reshape(1, 3 * D))

    nsv_g, so_g = _gather2_call(nsv_p, so, src)

    bm = 4000
    msg = pl.pallas_call(
        _m_body,
        grid=(E // bm,),
        in_specs=[
            pl.BlockSpec((bm, DE), lambda i: (i, 0)),
            pl.BlockSpec((bm, 1), lambda i: (i, 0)),
            pl.BlockSpec((DE, 3 * D), lambda i: (0, 0)),
            pl.BlockSpec((1, 3 * D), lambda i: (0, 0)),
            pl.BlockSpec((bm, 2 * D), lambda i: (i, 0)),
            pl.BlockSpec((bm, 2 * D), lambda i: (i, 0)),
            pl.BlockSpec((bm, 3), lambda i: (i, 0)),
        ],
        out_specs=pl.BlockSpec((4, bm, D), lambda i: (0, i, 0)),
        out_shape=jax.ShapeDtypeStruct((4, _EPAD, D), jnp.float32),
    )(edge_states, edge_norms, Wf, bf.reshape(1, 3 * D), so_g, nsv_g,
      edge_vectors)

    dp = _scatter_call(msg, dst_r, zeros_sub)

    outs, outv = pl.pallas_call(
        _b_body,
        grid=(N // bn,),
        in_specs=[
            pl.BlockSpec((bn, D), lambda i: (i, 0)),
            pl.BlockSpec((bn, 3 * D), lambda i: (i, 0)),
            pl.BlockSpec((4, bn, D), lambda i: (0, i, 0)),
            pl.BlockSpec((D, D), lambda i: (0, 0)),
            pl.BlockSpec((D, D), lambda i: (0, 0)),
            pl.BlockSpec((2 * D, D), lambda i: (0, 0)),
            pl.BlockSpec((1, D), lambda i: (0, 0)),
            pl.BlockSpec((D, 3 * D), lambda i: (0, 0)),
            pl.BlockSpec((1, 3 * D), lambda i: (0, 0)),
        ],
        out_specs=[
            pl.BlockSpec((bn, D), lambda i: (i, 0)),
            pl.BlockSpec((bn, 3 * D), lambda i: (i, 0)),
        ],
        out_shape=[
            jax.ShapeDtypeStruct((N, D), jnp.float32),
            jax.ShapeDtypeStruct((N, 3 * D), jnp.float32),
        ],
    )(ns, nsv_flat, dp, U, V, Wa1, ba1.reshape(1, D), Wa2,
      ba2.reshape(1, 3 * D))

    return outs, outv.reshape(N, 3, D)
